# baseline (speedup 1.0000x reference)
import jax
import jax.numpy as jnp
from jax import lax
from jax.experimental import pallas as pl
from jax.experimental.pallas import tpu as pltpu
from jax.experimental.pallas import tpu_sc as plsc

N_ENT = 50000
N_EDGES = 800000
D = 64
N_USERS = 1024
N_REL = 16

HALF = D // 2
NC = 2
NS = 16
N_PAD = 50048
ROWS_PER_TILE = N_PAD // NS
E_PER_TILE = N_EDGES // NS
CHUNK = 250
N_CHUNK = E_PER_TILE // CHUNK
SROWS = 10
NDS = N_CHUNK // (2 * SROWS)
TBL_ROWS = N_REL * N_PAD
TBL128 = NC * N_REL * N_PAD // 4
EROWS = N_EDGES // CHUNK


def _tb_body(emb_ref, w_ref, out_ref):
    row = pl.program_id(0) * N_REL + pl.program_id(1)
    out_ref[...] = emb_ref[...] * w_ref[pl.ds(row, 1), :]


_tb_call = pl.pallas_call(
    _tb_body,
    grid=(NC, N_REL),
    in_specs=[
        pl.BlockSpec((N_PAD * HALF // 128, 128), lambda c, r: (c, 0)),
        pl.BlockSpec((NC * N_REL, 128), lambda c, r: (0, 0)),
    ],
    out_specs=pl.BlockSpec((N_PAD * HALF // 128, 128),
                           lambda c, r: (c * N_REL + r, 0)),
    out_shape=jax.ShapeDtypeStruct((TBL128, 128), jnp.float32),
)


def _ci_body(tail_ref, type_ref, out_ref):
    out_ref[...] = type_ref[...] * N_PAD + tail_ref[...]


_ci_call = pl.pallas_call(
    _ci_body,
    out_shape=jax.ShapeDtypeStruct((N_EDGES,), jnp.int32),
)


def _sc_body(scaled, cidx2, head2, out2,
             acc, tb0, tb1, cs0, cs1, hs0, hs1,
             g0, g1, s0, s1, sts0, sts1):
    c = lax.axis_index("c")
    s = lax.axis_index("s")
    row0 = s * ROWS_PER_TILE
    rbase = s * N_CHUNK
    zero16 = jnp.zeros((16,), jnp.float32)
    src = scaled.at[c]

    def _wait_g(sem, buf):
        pltpu.make_async_copy(out2.at[0, pl.ds(0, CHUNK)], buf, sem).wait()

    def _wait_s(sem, buf):
        pltpu.make_async_copy(buf, acc.at[pl.ds(0, CHUNK)], sem).wait()

    def _wait_stage(sem, cb, hb):
        pltpu.make_async_copy(cidx2.at[pl.ds(0, SROWS)], cb, sem).wait()
        pltpu.make_async_copy(head2.at[pl.ds(0, SROWS)], hb, sem).wait()

    pltpu.async_copy(cidx2.at[pl.ds(rbase, SROWS)], cs0, sts0)
    pltpu.async_copy(head2.at[pl.ds(rbase, SROWS)], hs0, sts0)

    def _z(i, _):
        tb0[i, pl.ds(0, 16)] = zero16
        tb0[i, pl.ds(16, 16)] = zero16
        tb1[i, pl.ds(0, 16)] = zero16
        tb1[i, pl.ds(16, 16)] = zero16
        return 0
    lax.fori_loop(0, CHUNK, _z, 0)

    def _zcopy(k, _):
        pltpu.sync_copy(tb0, acc.at[pl.ds(row0 + k * CHUNK, CHUNK)])
        return 0
    lax.fori_loop(0, ROWS_PER_TILE // CHUNK, _zcopy, 0)
    pltpu.sync_copy(tb0.at[pl.ds(0, ROWS_PER_TILE % CHUNK)],
                    acc.at[pl.ds(row0 + (ROWS_PER_TILE // CHUNK) * CHUNK,
                                 ROWS_PER_TILE % CHUNK)])
    _wait_stage(sts0, cs0, hs0)
    plsc.subcore_barrier()

    pltpu.async_copy(src.at[cs0.at[0]], tb0, g0)
    pltpu.async_copy(tb1, acc.at[hs0.at[0]], s1, add=True)

    def _super(cb, hb, last_gather):
        for p in range(SROWS // 2):
            _wait_g(g0, tb0)
            pltpu.async_copy(tb0, acc.at[hb.at[2 * p]], s0, add=True)
            _wait_s(s1, tb1)
            pltpu.async_copy(src.at[cb.at[2 * p + 1]], tb1, g1)
            _wait_g(g1, tb1)
            pltpu.async_copy(tb1, acc.at[hb.at[2 * p + 1]], s1, add=True)
            _wait_s(s0, tb0)
            if p < SROWS // 2 - 1:
                pltpu.async_copy(src.at[cb.at[2 * p + 2]], tb0, g0)
            else:
                last_gather()

    def _dsuper(m, _):
        r1 = rbase + (2 * m + 1) * SROWS
        pltpu.async_copy(cidx2.at[pl.ds(r1, SROWS)], cs1, sts1)
        pltpu.async_copy(head2.at[pl.ds(r1, SROWS)], hs1, sts1)

        def _lg0():
            _wait_stage(sts1, cs1, hs1)
            pltpu.async_copy(src.at[cs1.at[0]], tb0, g0)
        _super(cs0, hs0, _lg0)

        r2 = jnp.where(m < NDS - 1, rbase + (2 * m + 2) * SROWS, rbase)
        pltpu.async_copy(cidx2.at[pl.ds(r2, SROWS)], cs0, sts0)
        pltpu.async_copy(head2.at[pl.ds(r2, SROWS)], hs0, sts0)

        def _lg1():
            _wait_stage(sts0, cs0, hs0)
            pltpu.async_copy(src.at[cs0.at[0]], tb0, g0)
        _super(cs1, hs1, _lg1)
        return 0
    lax.fori_loop(0, NDS, _dsuper, 0)

    _wait_g(g0, tb0)
    _wait_s(s1, tb1)

    plsc.subcore_barrier()
    pltpu.sync_copy(acc.at[pl.ds(row0, ROWS_PER_TILE)],
                    out2.at[c, pl.ds(row0, ROWS_PER_TILE)])


_sc_call = pl.kernel(
    _sc_body,
    out_type=jax.ShapeDtypeStruct((NC, N_PAD, HALF), jnp.float32),
    mesh=plsc.VectorSubcoreMesh(core_axis_name="c", subcore_axis_name="s",
                                num_cores=NC, num_subcores=NS),
    scratch_types=[
        pltpu.VMEM_SHARED((N_PAD, HALF), jnp.float32),
        pltpu.VMEM((CHUNK, HALF), jnp.float32),
        pltpu.VMEM((CHUNK, HALF), jnp.float32),
        pltpu.VMEM((SROWS, CHUNK), jnp.int32),
        pltpu.VMEM((SROWS, CHUNK), jnp.int32),
        pltpu.VMEM((SROWS, CHUNK), jnp.int32),
        pltpu.VMEM((SROWS, CHUNK), jnp.int32),
        pltpu.SemaphoreType.DMA,
        pltpu.SemaphoreType.DMA,
        pltpu.SemaphoreType.DMA,
        pltpu.SemaphoreType.DMA,
        pltpu.SemaphoreType.DMA,
        pltpu.SemaphoreType.DMA,
    ],
    compiler_params=pltpu.CompilerParams(use_tc_tiling_on_sc=False),
)


def _mm_body(emb_ref, imT_ref, out_ref):
    @pl.when(pl.program_id(0) == 0)
    def _():
        out_ref[...] = jnp.zeros_like(out_ref)

    out_ref[...] += lax.dot_general(
        emb_ref[...], imT_ref[...],
        dimension_numbers=(((0,), (0,)), ((), ())),
        preferred_element_type=jnp.float32)


_KBLK = 2000

_mm_call = pl.pallas_call(
    _mm_body,
    grid=(N_ENT // _KBLK,),
    in_specs=[
        pl.BlockSpec((_KBLK, D), lambda k: (k, 0)),
        pl.BlockSpec((_KBLK, N_USERS), lambda k: (k, 0)),
    ],
    out_specs=pl.BlockSpec((D, N_USERS), lambda k: (0, 0)),
    out_shape=jax.ShapeDtypeStruct((D, N_USERS), jnp.float32),
)


def kernel(entity_emb, user_emb, edge_index, edge_type, interact_mat, weight):
    pad = ((0, N_PAD - N_ENT), (0, 0))
    emb2p = jnp.concatenate([jnp.pad(entity_emb[:, :HALF], pad),
                             jnp.pad(entity_emb[:, HALF:], pad)], axis=0)
    emb128 = emb2p.reshape(NC * N_PAD * HALF // 128, 128)
    w4 = jnp.tile(jnp.concatenate([weight[:, :HALF], weight[:, HALF:]],
                                  axis=0), (1, 4))
    head = edge_index[0]
    tail = edge_index[1]

    scaled = _tb_call(emb128, w4).reshape(NC, TBL_ROWS, HALF)
    cidx2 = _ci_call(tail, edge_type).reshape(EROWS, CHUNK)
    head2 = head.reshape(EROWS, CHUNK)

    user_agg = _mm_call(entity_emb, interact_mat.T).T
    out2 = _sc_call(scaled, cidx2, head2)
    entity_agg = out2[:, :N_ENT, :].transpose(1, 0, 2).reshape(N_ENT, D)
    return (entity_agg, user_agg)


# speedup vs baseline: 1.0320x; 1.0320x over previous
import jax
import jax.numpy as jnp
from jax import lax
from jax.experimental import pallas as pl
from jax.experimental.pallas import tpu as pltpu
from jax.experimental.pallas import tpu_sc as plsc

N_ENT = 50000
N_EDGES = 800000
D = 64
N_USERS = 1024
N_REL = 16

HALF = D // 2
NC = 2
NS = 16
N_PAD = 50048
ROWS_PER_TILE = N_PAD // NS
E_PER_TILE = N_EDGES // NS
CHUNK = 250
N_CHUNK = E_PER_TILE // CHUNK
SROWS = 10
NDS = N_CHUNK // (2 * SROWS)
TBL_ROWS = N_REL * N_PAD
TBL128 = NC * N_REL * N_PAD // 4
EROWS = N_EDGES // CHUNK


def _tb_body(emb_ref, w_ref, out_ref):
    row = pl.program_id(0) * N_REL + pl.program_id(1)
    out_ref[...] = emb_ref[...] * w_ref[pl.ds(row, 1), :]


_tb_call = pl.pallas_call(
    _tb_body,
    grid=(NC, N_REL),
    in_specs=[
        pl.BlockSpec((N_PAD * HALF // 128, 128), lambda c, r: (c, 0)),
        pl.BlockSpec((NC * N_REL, 128), lambda c, r: (0, 0)),
    ],
    out_specs=pl.BlockSpec((N_PAD * HALF // 128, 128),
                           lambda c, r: (c * N_REL + r, 0)),
    out_shape=jax.ShapeDtypeStruct((TBL128, 128), jnp.float32),
)


def _ci_body(tail_ref, type_ref, out_ref):
    out_ref[...] = type_ref[...] * N_PAD + tail_ref[...]


_ci_call = pl.pallas_call(
    _ci_body,
    out_shape=jax.ShapeDtypeStruct((N_EDGES,), jnp.int32),
)


def _sc_body(scaled, cidx2, head2, out2,
             acc, tb0, tb1, cs0, cs1, hs0, hs1,
             g0, g1, s0, s1, sts0, sts1):
    c = lax.axis_index("c")
    s = lax.axis_index("s")
    row0 = s * ROWS_PER_TILE
    rbase = s * N_CHUNK
    zero16 = jnp.zeros((16,), jnp.float32)
    src = scaled.at[c]

    def _wait_g(sem, buf):
        pltpu.make_async_copy(out2.at[0, pl.ds(0, CHUNK)], buf, sem).wait()

    def _wait_s(sem, buf):
        pltpu.make_async_copy(buf, acc.at[pl.ds(0, CHUNK)], sem).wait()

    def _wait_stage(sem, cb, hb):
        pltpu.make_async_copy(cidx2.at[pl.ds(0, SROWS)], cb, sem).wait()
        pltpu.make_async_copy(head2.at[pl.ds(0, SROWS)], hb, sem).wait()

    pltpu.async_copy(cidx2.at[pl.ds(rbase, SROWS)], cs0, sts0)
    pltpu.async_copy(head2.at[pl.ds(rbase, SROWS)], hs0, sts0)

    def _z(i, _):
        tb0[i, pl.ds(0, 16)] = zero16
        tb0[i, pl.ds(16, 16)] = zero16
        tb1[i, pl.ds(0, 16)] = zero16
        tb1[i, pl.ds(16, 16)] = zero16
        return 0
    lax.fori_loop(0, CHUNK, _z, 0)

    def _zcopy(k, _):
        pltpu.sync_copy(tb0, acc.at[pl.ds(row0 + k * CHUNK, CHUNK)])
        return 0
    lax.fori_loop(0, ROWS_PER_TILE // CHUNK, _zcopy, 0)
    pltpu.sync_copy(tb0.at[pl.ds(0, ROWS_PER_TILE % CHUNK)],
                    acc.at[pl.ds(row0 + (ROWS_PER_TILE // CHUNK) * CHUNK,
                                 ROWS_PER_TILE % CHUNK)])
    _wait_stage(sts0, cs0, hs0)
    plsc.subcore_barrier()

    pltpu.async_copy(src.at[cs0.at[0]], tb0, g0)
    pltpu.async_copy(tb1, acc.at[hs0.at[0]], s1, add=True)

    def _super(cb, hb, last_gather):
        for p in range(SROWS // 2):
            _wait_g(g0, tb0)
            pltpu.async_copy(tb0, acc.at[hb.at[2 * p]], s0, add=True)
            _wait_s(s1, tb1)
            pltpu.async_copy(src.at[cb.at[2 * p + 1]], tb1, g1)
            _wait_g(g1, tb1)
            pltpu.async_copy(tb1, acc.at[hb.at[2 * p + 1]], s1, add=True)
            _wait_s(s0, tb0)
            if p < SROWS // 2 - 1:
                pltpu.async_copy(src.at[cb.at[2 * p + 2]], tb0, g0)
            else:
                last_gather()

    def _dsuper(m, _):
        r1 = rbase + (2 * m + 1) * SROWS
        pltpu.async_copy(cidx2.at[pl.ds(r1, SROWS)], cs1, sts1)
        pltpu.async_copy(head2.at[pl.ds(r1, SROWS)], hs1, sts1)

        def _lg0():
            _wait_stage(sts1, cs1, hs1)
            pltpu.async_copy(src.at[cs1.at[0]], tb0, g0)
        _super(cs0, hs0, _lg0)

        r2 = jnp.where(m < NDS - 1, rbase + (2 * m + 2) * SROWS, rbase)
        pltpu.async_copy(cidx2.at[pl.ds(r2, SROWS)], cs0, sts0)
        pltpu.async_copy(head2.at[pl.ds(r2, SROWS)], hs0, sts0)

        def _lg1():
            _wait_stage(sts0, cs0, hs0)
            pltpu.async_copy(src.at[cs0.at[0]], tb0, g0)
        _super(cs1, hs1, _lg1)
        return 0
    lax.fori_loop(0, NDS, _dsuper, 0)

    _wait_g(g0, tb0)
    _wait_s(s1, tb1)

    plsc.subcore_barrier()
    pltpu.sync_copy(acc.at[pl.ds(row0, ROWS_PER_TILE)],
                    out2.at[c, pl.ds(row0, ROWS_PER_TILE)])


_sc_call = pl.kernel(
    _sc_body,
    out_type=jax.ShapeDtypeStruct((NC, N_PAD, HALF), jnp.float32),
    mesh=plsc.VectorSubcoreMesh(core_axis_name="c", subcore_axis_name="s",
                                num_cores=NC, num_subcores=NS),
    scratch_types=[
        pltpu.VMEM_SHARED((N_PAD, HALF), jnp.float32),
        pltpu.VMEM((CHUNK, HALF), jnp.float32),
        pltpu.VMEM((CHUNK, HALF), jnp.float32),
        pltpu.VMEM((SROWS, CHUNK), jnp.int32),
        pltpu.VMEM((SROWS, CHUNK), jnp.int32),
        pltpu.VMEM((SROWS, CHUNK), jnp.int32),
        pltpu.VMEM((SROWS, CHUNK), jnp.int32),
        pltpu.SemaphoreType.DMA,
        pltpu.SemaphoreType.DMA,
        pltpu.SemaphoreType.DMA,
        pltpu.SemaphoreType.DMA,
        pltpu.SemaphoreType.DMA,
        pltpu.SemaphoreType.DMA,
    ],
    compiler_params=pltpu.CompilerParams(use_tc_tiling_on_sc=False),
)


_KBLK = 1920
_NKB = -(-N_ENT // _KBLK)


def _mm_body(embT_ref, imT_ref, out_ref):
    k = pl.program_id(0)

    @pl.when(k == 0)
    def _():
        out_ref[...] = jnp.zeros_like(out_ref)

    a = embT_ref[...]
    lanes = lax.broadcasted_iota(jnp.int32, a.shape, 1) + k * _KBLK
    a = jnp.where(lanes < N_ENT, a, 0.0)
    out_ref[...] += lax.dot_general(
        a, imT_ref[...],
        dimension_numbers=(((1,), (0,)), ((), ())),
        preferred_element_type=jnp.float32)


_mm_call = pl.pallas_call(
    _mm_body,
    grid=(_NKB,),
    in_specs=[
        pl.BlockSpec((D, _KBLK), lambda k: (0, k)),
        pl.BlockSpec((_KBLK, N_USERS), lambda k: (k, 0)),
    ],
    out_specs=pl.BlockSpec((D, N_USERS), lambda k: (0, 0)),
    out_shape=jax.ShapeDtypeStruct((D, N_USERS), jnp.float32),
)


def kernel(entity_emb, user_emb, edge_index, edge_type, interact_mat, weight):
    pad = ((0, N_PAD - N_ENT), (0, 0))
    emb2p = jnp.concatenate([jnp.pad(entity_emb[:, :HALF], pad),
                             jnp.pad(entity_emb[:, HALF:], pad)], axis=0)
    emb128 = emb2p.reshape(NC * N_PAD * HALF // 128, 128)
    w4 = jnp.tile(jnp.concatenate([weight[:, :HALF], weight[:, HALF:]],
                                  axis=0), (1, 4))
    head = edge_index[0]
    tail = edge_index[1]

    scaled = _tb_call(emb128, w4).reshape(NC, TBL_ROWS, HALF)
    cidx2 = _ci_call(tail, edge_type).reshape(EROWS, CHUNK)
    head2 = head.reshape(EROWS, CHUNK)

    user_agg = _mm_call(entity_emb.T, interact_mat.T).T
    out2 = _sc_call(scaled, cidx2, head2)
    entity_agg = out2[:, :N_ENT, :].transpose(1, 0, 2).reshape(N_ENT, D)
    return (entity_agg, user_agg)
